# baseline (speedup 1.0000x reference)
import jax
import jax.numpy as jnp
from jax import lax
from jax.experimental import pallas as pl
from jax.experimental.pallas import tpu as pltpu
from jax.experimental.pallas import tpu_sc as plsc

NUM_WORKERS = 32
CHUNK = 128
VOCAB = 1000000
D = 64
TOTAL = 4096 * 200
PER_W = TOTAL // NUM_WORKERS
NCH = PER_W // CHUNK
NBUF = 4


def _body(ids_hbm, table_hbm, out_hbm, idx_v, bufs, sem_g, sem_w):
    wid = lax.axis_index("s") * 2 + lax.axis_index("c")
    base = wid * PER_W

    pltpu.sync_copy(ids_hbm.at[wid], idx_v)

    def gather(j, b):
        pltpu.async_copy(table_hbm.at[idx_v.at[j]], bufs.at[b], sem_g)

    def writeback(j, b):
        pltpu.async_copy(bufs.at[b], out_hbm.at[pl.ds(base + j * CHUNK, CHUNK)],
                         sem_w)

    def wait_g():
        pltpu.make_async_copy(table_hbm.at[idx_v.at[0]], bufs.at[0],
                              sem_g).wait()

    def wait_w():
        pltpu.make_async_copy(bufs.at[0], out_hbm.at[pl.ds(base, CHUNK)],
                              sem_w).wait()

    gather(0, 0)
    gather(1, 1)
    for b in range(NBUF):
        if b >= 2:
            wait_w()
        gather(b + 2, (b + 2) % NBUF)
        wait_g()
        writeback(b, b)

    def step(g, carry):
        j0 = g * NBUF
        for b in range(NBUF):
            wait_w()
            gather(j0 + b + 2, (b + 2) % NBUF)
            wait_g()
            writeback(j0 + b, b)
        return carry

    lax.fori_loop(1, NCH // NBUF - 1, step, 0)

    j0 = NCH - NBUF
    for b in range(NBUF):
        wait_w()
        if b < 2:
            gather(j0 + b + 2, (b + 2) % NBUF)
        wait_g()
        writeback(j0 + b, b)
    wait_w()
    wait_w()


def kernel(token_ids, weights):
    ids = token_ids.reshape(NUM_WORKERS, NCH, CHUNK).astype(jnp.int32)
    run = pl.kernel(
        _body,
        out_type=jax.ShapeDtypeStruct((TOTAL, D), jnp.float32),
        mesh=plsc.VectorSubcoreMesh(core_axis_name="c", subcore_axis_name="s"),
        scratch_types=[
            pltpu.VMEM((NCH, CHUNK), jnp.int32),
            pltpu.VMEM((NBUF, CHUNK, D), jnp.float32),
            pltpu.SemaphoreType.DMA,
            pltpu.SemaphoreType.DMA,
        ],
        compiler_params=pltpu.CompilerParams(use_tc_tiling_on_sc=False),
    )
    out = run(ids, weights)
    return out.reshape(4096, 200, D)
